# baseline (speedup 1.0000x reference)
import functools
import math

import jax
import jax.numpy as jnp
from jax import lax
from jax.experimental import pallas as pl
from jax.experimental.pallas import tpu as pltpu
from jax.experimental.pallas import tpu_sc as plsc

_NUM_CLASSES = 80
_IMG_SIZE = 512.0
_LAMBDA_BOX = 7.5
_LAMBDA_CLS = 0.5
_TOPK = 10
_ALPHA_COST = 1.5
_BETA_COST = 6.0
_ALPHA = 0.25
_EPS = 1e-7

_F32 = jnp.float32
_I32 = jnp.int32


def _sigmoid(x):
    return 1.0 / (1.0 + jnp.exp(-x))


def _softplus(x):
    return jnp.maximum(x, 0.0) + jnp.log(1.0 + jnp.exp(-jnp.abs(x)))


def _atan_pos(x):
    t3 = x > 2.414213562373095
    t1 = x > 0.4142135623730950
    xr = jnp.where(t3, -1.0 / x, jnp.where(t1, (x - 1.0) / (x + 1.0), x))
    y = jnp.where(t3, math.pi / 2.0, jnp.where(t1, math.pi / 4.0, 0.0))
    z = xr * xr
    p = ((8.05374449538e-2 * z - 1.38776856032e-1) * z + 1.99777106478e-1) * z \
        - 3.33329491539e-1
    return y + xr + xr * z * p


def _focal_body(preds_ref, fneg_ref):
    E = 4 + _NUM_CLASSES
    l = preds_ref[0][:, 4:E]
    p = _sigmoid(l)
    ce0 = _softplus(l)
    fneg = (1.0 - _ALPHA) * ce0 * p * p
    fneg_ref[0] = jnp.full((1, 128), jnp.sum(fneg), dtype=_F32)


def _focal_call(preds_r):
    B, N, E = preds_r.shape
    return pl.pallas_call(
        _focal_body,
        grid=(B,),
        in_specs=[pl.BlockSpec((1, N, E), lambda i: (i, 0, 0))],
        out_specs=pl.BlockSpec((1, 1, 128), lambda i: (i, 0, 0)),
        out_shape=jax.ShapeDtypeStruct((B, 1, 128), _F32),
    )(preds_r)


def _head_body(S, G, preds_ref, gtb_ref, lbl_ref, vals_ref, ids_ref):
    N = S * S
    E = 4 + _NUM_CLASSES
    stride = _IMG_SIZE / float(S)
    blk = preds_ref[0]
    gtb = gtb_ref[0]
    lbl = lbl_ref[0]

    e4 = (lax.broadcasted_iota(_I32, (4, E), 0)
          == lax.broadcasted_iota(_I32, (4, E), 1)).astype(_F32)
    boxT = lax.dot_general(e4, blk, (((1,), (1,)), ((), ())),
                           preferred_element_type=_F32)
    tx, ty, tw, th = boxT[0:1], boxT[1:2], boxT[2:3], boxT[3:4]
    n = lax.broadcasted_iota(_I32, (1, N), 1)
    gxf = (n % S).astype(_F32)
    gyf = (n // S).astype(_F32)
    px = (_sigmoid(tx) * 2.0 - 0.5 + gxf) * stride
    py = (_sigmoid(ty) * 2.0 - 0.5 + gyf) * stride
    pw = _softplus(tw) * stride
    ph = _softplus(th) * stride
    px1 = px - 0.5 * pw
    px2 = px + 0.5 * pw
    py1 = py - 0.5 * ph
    py2 = py + 0.5 * ph

    gcx = gtb[:, 0:1] * _IMG_SIZE
    gcy = gtb[:, 1:2] * _IMG_SIZE
    gw = gtb[:, 2:3] * _IMG_SIZE
    gh = gtb[:, 3:4] * _IMG_SIZE
    gx1 = gcx - 0.5 * gw
    gx2 = gcx + 0.5 * gw
    gy1 = gcy - 0.5 * gh
    gy2 = gcy + 0.5 * gh

    a1 = jnp.maximum(px2 - px1, 0.0) * jnp.maximum(py2 - py1, 0.0)
    a2 = jnp.maximum(gx2 - gx1, 0.0) * jnp.maximum(gy2 - gy1, 0.0)
    ix1 = jnp.maximum(px1, gx1)
    iy1 = jnp.maximum(py1, gy1)
    ix2 = jnp.minimum(px2, gx2)
    iy2 = jnp.minimum(py2, gy2)
    inter = jnp.maximum(ix2 - ix1, 0.0) * jnp.maximum(iy2 - iy1, 0.0)
    union = a1 + a2 - inter + _EPS
    iou = jnp.clip(inter / union, 0.0, 1.0)

    oh = (lax.broadcasted_iota(_I32, (G, E), 1) == (lbl + 4)).astype(_F32)
    selT = lax.dot_general(oh, blk, (((1,), (1,)), ((), ())),
                           preferred_element_type=_F32)
    pc = jnp.clip(_sigmoid(selT), 1e-6, 1.0 - 1e-6)
    cost = _ALPHA_COST * (-jnp.log(pc)) + _BETA_COST * (-jnp.log(iou + _EPS))

    INF = _F32(jnp.inf)
    lane = lax.broadcasted_iota(_I32, (G, N), 1)
    gidx = lax.broadcasted_iota(_I32, (G, 1), 0)
    work = cost
    candm = jnp.zeros((G, N), dtype=_F32)
    for t in range(_TOPK):
        m = jnp.min(work, axis=1, keepdims=True)
        idx = jnp.min(jnp.where(work == m, lane, _I32(1 << 30)),
                      axis=1, keepdims=True)
        vals_ref[0, :, t:t + 1] = m
        ids_ref[0, :, t:t + 1] = idx
        hit = lane == idx
        candm = jnp.where(hit, 1.0, candm)
        work = jnp.where(hit, INF, work)
    cand = jnp.any(candm > 0.0, axis=0, keepdims=True)

    masked = jnp.where(cand, work, INF)
    for t in range(_TOPK, G):
        m = jnp.min(masked, axis=1, keepdims=True)
        fin = m < _F32(1e30)
        idx = jnp.min(jnp.where(masked == m, lane, _I32(1 << 30)),
                      axis=1, keepdims=True)
        vals_ref[0, :, t:t + 1] = jnp.where(fin, m, INF)
        ids_ref[0, :, t:t + 1] = jnp.where(fin, idx, -2 - (gidx * G + t))
        masked = jnp.where(lane == idx, INF, masked)


def _head_call(preds_r, gt_boxes, lbl_r):
    B, N, E = preds_r.shape
    S = int(round(math.sqrt(N)))
    G = gt_boxes.shape[1]
    body = functools.partial(_head_body, S, G)
    out_shape = [jax.ShapeDtypeStruct((B, G, G), _F32),
                 jax.ShapeDtypeStruct((B, G, G), _I32)]
    spec_gg = pl.BlockSpec((1, G, G), lambda i: (i, 0, 0))
    return pl.pallas_call(
        body,
        grid=(B,),
        in_specs=[pl.BlockSpec((1, N, E), lambda i: (i, 0, 0)),
                  pl.BlockSpec((1, G, 4), lambda i: (i, 0, 0)),
                  pl.BlockSpec((1, G, 1), lambda i: (i, 0, 0))],
        out_specs=[spec_gg, spec_gg],
        out_shape=out_shape,
    )(preds_r, gt_boxes, lbl_r)


def _greedy_sc_body(B, G, vals_hbm, ids_hbm, sp_out, vals_v, ids_v, sp_v):
    P = G * G
    NCH = P // 16
    c = lax.axis_index("c")
    s = lax.axis_index("s")
    w = s * 2 + c

    @pl.when(w < B)
    def _():
        pltpu.sync_copy(vals_hbm.at[pl.ds(w * P, P)], vals_v)
        pltpu.sync_copy(ids_hbm.at[pl.ds(w * P, P)], ids_v)
        lanes = lax.iota(_I32, 16)
        BIGI = _I32(1 << 30)
        sp_v[pl.ds(0, 16)] = jnp.full((16,), -1, dtype=_I32)
        sp_v[pl.ds(16, 16)] = jnp.full((16,), -1, dtype=_I32)
        sp_v[pl.ds(32, 16)] = jnp.full((16,), 0, dtype=_I32)
        sp_v[pl.ds(48, 16)] = jnp.full((16,), 0, dtype=_I32)

        def iter_body(t, carry):
            mv = jnp.full((16,), jnp.inf, dtype=_F32)
            pv = jnp.full((16,), BIGI, dtype=_I32)
            for ci in range(NCH):
                v = vals_v[pl.ds(ci * 16, 16)]
                pos = lanes + ci * 16
                better = v < mv
                mv = jnp.where(better, v, mv)
                pv = jnp.where(better, pos, pv)
            m = mv[0]
            for i in range(1, 16):
                m = jnp.minimum(m, mv[i])
            pw = jnp.where(mv == jnp.full((16,), m, dtype=_F32), pv, BIGI)
            pos = pw[0]
            for i in range(1, 16):
                pos = jnp.minimum(pos, pw[i])
            posB = jnp.full((16,), pos, dtype=_I32)
            idm = jnp.full((16,), BIGI, dtype=_I32)
            for ci in range(NCH):
                vi = ids_v[pl.ds(ci * 16, 16)]
                posv = lanes + ci * 16
                idm = jnp.minimum(idm, jnp.where(posv == posB, vi, BIGI))
            pid = idm[0]
            for i in range(1, 16):
                pid = jnp.minimum(pid, idm[i])
            g = pos // G
            slot = pos - g * G
            gv = jnp.full((16,), g, dtype=_I32)
            pidv = jnp.full((16,), pid, dtype=_I32)
            slotv = jnp.full((16,), slot, dtype=_I32)
            lo = jnp.full((16,), g * G, dtype=_I32)
            hi = jnp.full((16,), g * G + G, dtype=_I32)
            for ci in range(NCH):
                v = vals_v[pl.ds(ci * 16, 16)]
                vi = ids_v[pl.ds(ci * 16, 16)]
                posv = lanes + ci * 16
                kill = (vi == pidv) | ((posv >= lo) & (posv < hi))
                vals_v[pl.ds(ci * 16, 16)] = jnp.where(kill, jnp.inf, v)
            a0 = sp_v[pl.ds(0, 16)]
            sp_v[pl.ds(0, 16)] = jnp.where(lanes == gv, slotv, a0)
            a1 = sp_v[pl.ds(16, 16)]
            sp_v[pl.ds(16, 16)] = jnp.where((lanes + 16) == gv, slotv, a1)
            p0 = sp_v[pl.ds(32, 16)]
            sp_v[pl.ds(32, 16)] = jnp.where(lanes == gv, pidv, p0)
            p1 = sp_v[pl.ds(48, 16)]
            sp_v[pl.ds(48, 16)] = jnp.where((lanes + 16) == gv, pidv, p1)
            return carry

        lax.fori_loop(0, G, iter_body, 0)
        pltpu.sync_copy(sp_v, sp_out.at[pl.ds(w * 64, 64)])


def _greedy_call(vals_flat, ids_flat, B, G):
    mesh = plsc.VectorSubcoreMesh(core_axis_name="c", subcore_axis_name="s")
    body = functools.partial(_greedy_sc_body, B, G)
    kern = pl.kernel(
        body,
        out_type=jax.ShapeDtypeStruct((B * 64,), _I32),
        mesh=mesh,
        scratch_types=[pltpu.VMEM((G * G,), _F32),
                       pltpu.VMEM((G * G,), _I32),
                       pltpu.VMEM((64,), _I32)],
    )
    return kern(vals_flat, ids_flat)


def _final_body(B, G, S, sp_ref, preds_ref, gtb_ref, lbl_ref, fneg_ref,
                out_ref, acc):
    N = S * S
    E = 4 + _NUM_CLASSES
    stride = _IMG_SIZE / float(S)
    i = pl.program_id(0)
    blk = preds_ref[0]
    sp = sp_ref[0]
    slot = sp[0:G, 0:1]
    pid = sp[32:32 + G, 0:1]
    valid = slot >= 0
    lbl = lbl_ref[0]

    oh = (lax.broadcasted_iota(_I32, (G, N), 1) == pid).astype(_F32)
    rows = lax.dot_general(oh, blk, (((1,), (0,)), ((), ())),
                           preferred_element_type=_F32)
    tx = rows[:, 0:1]
    ty = rows[:, 1:2]
    tw_r = rows[:, 2:3]
    th_r = rows[:, 3:4]
    ohl = (lax.broadcasted_iota(_I32, (G, E), 1) == (lbl + 4)).astype(_F32)
    msel = jnp.sum(rows * ohl, axis=1, keepdims=True)

    gxf = (pid % S).astype(_F32)
    gyf = (pid // S).astype(_F32)
    px = (_sigmoid(tx) * 2.0 - 0.5 + gxf) * stride
    py = (_sigmoid(ty) * 2.0 - 0.5 + gyf) * stride
    pwd = _softplus(tw_r) * stride
    phd = _softplus(th_r) * stride
    mx1 = px - 0.5 * pwd
    mx2 = px + 0.5 * pwd
    my1 = py - 0.5 * phd
    my2 = py + 0.5 * phd

    gtb = gtb_ref[0]
    gcx = gtb[:, 0:1] * _IMG_SIZE
    gcy = gtb[:, 1:2] * _IMG_SIZE
    gw = gtb[:, 2:3] * _IMG_SIZE
    gh = gtb[:, 3:4] * _IMG_SIZE
    gx1 = gcx - 0.5 * gw
    gx2 = gcx + 0.5 * gw
    gy1 = gcy - 0.5 * gh
    gy2 = gcy + 0.5 * gh

    pw = jnp.maximum(mx2 - mx1, _EPS)
    ph = jnp.maximum(my2 - my1, _EPS)
    tw = jnp.maximum(gx2 - gx1, _EPS)
    th = jnp.maximum(gy2 - gy1, _EPS)
    iw = jnp.maximum(jnp.minimum(mx2, gx2) - jnp.maximum(mx1, gx1), 0.0)
    ih = jnp.maximum(jnp.minimum(my2, gy2) - jnp.maximum(my1, gy1), 0.0)
    inter = iw * ih
    union = pw * ph + tw * th - inter + _EPS
    iou = inter / union
    pcx = (mx1 + mx2) * 0.5
    pcy = (my1 + my2) * 0.5
    tcx = (gx1 + gx2) * 0.5
    tcy = (gy1 + gy2) * 0.5
    cd = (pcx - tcx) ** 2 + (pcy - tcy) ** 2
    cw = jnp.maximum(mx2, gx2) - jnp.minimum(mx1, gx1)
    ch = jnp.maximum(my2, gy2) - jnp.minimum(my1, gy1)
    c2 = cw ** 2 + ch ** 2 + _EPS
    v = (4.0 / (math.pi ** 2)) * (_atan_pos(tw / th) - _atan_pos(pw / ph)) ** 2
    alpha = v / (v - iou + 1.0 + _EPS)
    ciou = iou - cd / c2 - alpha * v
    box_i = jnp.sum(jnp.where(valid, 1.0 - ciou, 0.0))

    lm = msel
    pm = _sigmoid(lm)
    ce0 = _softplus(lm)
    ce1 = ce0 - lm
    delta = _ALPHA * ce1 * (1.0 - pm) ** 2 - (1.0 - _ALPHA) * ce0 * pm ** 2
    cls_i = fneg_ref[0][0, 0] + jnp.sum(jnp.where(valid, delta, 0.0))
    m_i = jnp.sum(jnp.where(valid, 1.0, 0.0))

    @pl.when(i == 0)
    def _():
        acc[0] = 0.0
        acc[1] = 0.0
        acc[2] = 0.0

    acc[0] += box_i
    acc[1] += cls_i
    acc[2] += m_i

    @pl.when(i == B - 1)
    def _():
        denom = jnp.maximum(acc[2], 1.0)
        loss = (_LAMBDA_BOX * acc[0] + _LAMBDA_CLS * acc[1]) / denom
        out_ref[...] = jnp.full((1, 128), loss, dtype=_F32)


def _final_call(sp3, preds_r, gt_boxes, lbl_r, fneg, S):
    B, G, _ = gt_boxes.shape
    N, E = preds_r.shape[1], preds_r.shape[2]
    body = functools.partial(_final_body, B, G, S)
    return pl.pallas_call(
        body,
        grid=(B,),
        in_specs=[pl.BlockSpec((1, 64, 1), lambda i: (i, 0, 0)),
                  pl.BlockSpec((1, N, E), lambda i: (i, 0, 0)),
                  pl.BlockSpec((1, G, 4), lambda i: (i, 0, 0)),
                  pl.BlockSpec((1, G, 1), lambda i: (i, 0, 0)),
                  pl.BlockSpec((1, 1, 128), lambda i: (i, 0, 0))],
        out_specs=pl.BlockSpec((1, 128), lambda i: (0, 0)),
        out_shape=jax.ShapeDtypeStruct((1, 128), _F32),
        scratch_shapes=[pltpu.SMEM((8,), _F32)],
    )(sp3, preds_r, gt_boxes, lbl_r, fneg)


def kernel(preds, gt_boxes, gt_labels):
    B, A, S, S2, E4 = preds.shape
    N = A * S * S2
    E = E4
    G = gt_boxes.shape[1]
    preds_r = preds.reshape(B, N, E)
    lbl_r = gt_labels.astype(_I32).reshape(B, G, 1)
    vals, ids = _head_call(preds_r, gt_boxes, lbl_r)
    sp = _greedy_call(vals.reshape(-1), ids.reshape(-1), B, G)
    fneg = _focal_call(preds_r)
    out = _final_call(sp.reshape(B, 64, 1), preds_r, gt_boxes, lbl_r, fneg, S)
    return out[0, 0]


# speedup vs baseline: 1.1028x; 1.1028x over previous
import functools
import math

import jax
import jax.numpy as jnp
from jax import lax
from jax.experimental import pallas as pl
from jax.experimental.pallas import tpu as pltpu
from jax.experimental.pallas import tpu_sc as plsc

_NUM_CLASSES = 80
_IMG_SIZE = 512.0
_LAMBDA_BOX = 7.5
_LAMBDA_CLS = 0.5
_TOPK = 10
_ALPHA_COST = 1.5
_BETA_COST = 6.0
_ALPHA = 0.25
_EPS = 1e-7

_F32 = jnp.float32
_I32 = jnp.int32


def _sigmoid(x):
    return 1.0 / (1.0 + jnp.exp(-x))


def _softplus(x):
    return jnp.maximum(x, 0.0) + jnp.log(1.0 + jnp.exp(-jnp.abs(x)))


def _atan_pos(x):
    t3 = x > 2.414213562373095
    t1 = x > 0.4142135623730950
    xr = jnp.where(t3, -1.0 / x, jnp.where(t1, (x - 1.0) / (x + 1.0), x))
    y = jnp.where(t3, math.pi / 2.0, jnp.where(t1, math.pi / 4.0, 0.0))
    z = xr * xr
    p = ((8.05374449538e-2 * z - 1.38776856032e-1) * z + 1.99777106478e-1) * z \
        - 3.33329491539e-1
    return y + xr + xr * z * p


def _head_body(S, G, preds_ref, gtb_ref, lbl_ref, vals_ref, ids_ref, fneg_ref):
    N = S * S
    E = 4 + _NUM_CLASSES
    stride = _IMG_SIZE / float(S)
    blk = preds_ref[0]
    gtb = gtb_ref[0]
    lbl = lbl_ref[0]

    l = blk[:, 4:E]
    p = _sigmoid(l)
    ce0 = _softplus(l)
    fneg = (1.0 - _ALPHA) * ce0 * p * p
    fneg_ref[0] = jnp.full((1, 128), jnp.sum(fneg), dtype=_F32)

    e4 = (lax.broadcasted_iota(_I32, (4, E), 0)
          == lax.broadcasted_iota(_I32, (4, E), 1)).astype(_F32)
    boxT = lax.dot_general(e4, blk, (((1,), (1,)), ((), ())),
                           preferred_element_type=_F32)
    tx, ty, tw, th = boxT[0:1], boxT[1:2], boxT[2:3], boxT[3:4]
    n = lax.broadcasted_iota(_I32, (1, N), 1)
    gxf = (n % S).astype(_F32)
    gyf = (n // S).astype(_F32)
    px = (_sigmoid(tx) * 2.0 - 0.5 + gxf) * stride
    py = (_sigmoid(ty) * 2.0 - 0.5 + gyf) * stride
    pw = _softplus(tw) * stride
    ph = _softplus(th) * stride
    px1 = px - 0.5 * pw
    px2 = px + 0.5 * pw
    py1 = py - 0.5 * ph
    py2 = py + 0.5 * ph

    gcx = gtb[:, 0:1] * _IMG_SIZE
    gcy = gtb[:, 1:2] * _IMG_SIZE
    gw = gtb[:, 2:3] * _IMG_SIZE
    gh = gtb[:, 3:4] * _IMG_SIZE
    gx1 = gcx - 0.5 * gw
    gx2 = gcx + 0.5 * gw
    gy1 = gcy - 0.5 * gh
    gy2 = gcy + 0.5 * gh

    a1 = jnp.maximum(px2 - px1, 0.0) * jnp.maximum(py2 - py1, 0.0)
    a2 = jnp.maximum(gx2 - gx1, 0.0) * jnp.maximum(gy2 - gy1, 0.0)
    ix1 = jnp.maximum(px1, gx1)
    iy1 = jnp.maximum(py1, gy1)
    ix2 = jnp.minimum(px2, gx2)
    iy2 = jnp.minimum(py2, gy2)
    inter = jnp.maximum(ix2 - ix1, 0.0) * jnp.maximum(iy2 - iy1, 0.0)
    union = a1 + a2 - inter + _EPS
    iou = jnp.clip(inter / union, 0.0, 1.0)

    oh = (lax.broadcasted_iota(_I32, (G, E), 1) == (lbl + 4)).astype(_F32)
    selT = lax.dot_general(oh, blk, (((1,), (1,)), ((), ())),
                           preferred_element_type=_F32)
    pc = jnp.clip(_sigmoid(selT), 1e-6, 1.0 - 1e-6)
    cost = _ALPHA_COST * (-jnp.log(pc)) + _BETA_COST * (-jnp.log(iou + _EPS))

    INF = _F32(jnp.inf)
    lane = lax.broadcasted_iota(_I32, (G, N), 1)
    gidx = lax.broadcasted_iota(_I32, (G, 1), 0)
    work = cost
    candm = jnp.zeros((G, N), dtype=_F32)
    for t in range(_TOPK):
        m = jnp.min(work, axis=1, keepdims=True)
        idx = jnp.min(jnp.where(work == m, lane, _I32(1 << 30)),
                      axis=1, keepdims=True)
        vals_ref[0, :, t:t + 1] = m
        ids_ref[0, :, t:t + 1] = idx
        hit = lane == idx
        candm = jnp.where(hit, 1.0, candm)
        work = jnp.where(hit, INF, work)
    cand = jnp.any(candm > 0.0, axis=0, keepdims=True)

    masked = jnp.where(cand, work, INF)
    for t in range(_TOPK, G):
        m = jnp.min(masked, axis=1, keepdims=True)
        fin = m < _F32(1e30)
        idx = jnp.min(jnp.where(masked == m, lane, _I32(1 << 30)),
                      axis=1, keepdims=True)
        vals_ref[0, :, t:t + 1] = jnp.where(fin, m, INF)
        ids_ref[0, :, t:t + 1] = jnp.where(fin, idx, -2 - (gidx * G + t))
        masked = jnp.where(lane == idx, INF, masked)


def _head_call(preds_r, gt_boxes, lbl_r):
    B, N, E = preds_r.shape
    S = int(round(math.sqrt(N)))
    G = gt_boxes.shape[1]
    body = functools.partial(_head_body, S, G)
    out_shape = [jax.ShapeDtypeStruct((B, G, G), _F32),
                 jax.ShapeDtypeStruct((B, G, G), _I32),
                 jax.ShapeDtypeStruct((B, 1, 128), _F32)]
    spec_gg = pl.BlockSpec((1, G, G), lambda i: (i, 0, 0))
    return pl.pallas_call(
        body,
        grid=(B,),
        in_specs=[pl.BlockSpec((1, N, E), lambda i: (i, 0, 0)),
                  pl.BlockSpec((1, G, 4), lambda i: (i, 0, 0)),
                  pl.BlockSpec((1, G, 1), lambda i: (i, 0, 0))],
        out_specs=[spec_gg, spec_gg, pl.BlockSpec((1, 1, 128), lambda i: (i, 0, 0))],
        out_shape=out_shape,
    )(preds_r, gt_boxes, lbl_r)


def _greedy_sc_body(B, G, vals_hbm, ids_hbm, sp_out, vals_v, ids_v, sp_v):
    P = G * G
    NCH = P // 16
    c = lax.axis_index("c")
    s = lax.axis_index("s")
    w = s * 2 + c

    @pl.when(w < B)
    def _():
        pltpu.sync_copy(vals_hbm.at[pl.ds(w * P, P)], vals_v)
        pltpu.sync_copy(ids_hbm.at[pl.ds(w * P, P)], ids_v)
        lanes = lax.iota(_I32, 16)
        BIGI = _I32(1 << 30)
        sp_v[pl.ds(0, 16)] = jnp.full((16,), -1, dtype=_I32)
        sp_v[pl.ds(16, 16)] = jnp.full((16,), -1, dtype=_I32)
        sp_v[pl.ds(32, 16)] = jnp.full((16,), 0, dtype=_I32)
        sp_v[pl.ds(48, 16)] = jnp.full((16,), 0, dtype=_I32)

        def iter_body(t, carry):
            mv = jnp.full((16,), jnp.inf, dtype=_F32)
            pv = jnp.full((16,), BIGI, dtype=_I32)
            for ci in range(NCH):
                v = vals_v[pl.ds(ci * 16, 16)]
                pos = lanes + ci * 16
                better = v < mv
                mv = jnp.where(better, v, mv)
                pv = jnp.where(better, pos, pv)
            m = mv[0]
            for i in range(1, 16):
                m = jnp.minimum(m, mv[i])
            pw = jnp.where(mv == jnp.full((16,), m, dtype=_F32), pv, BIGI)
            pos = pw[0]
            for i in range(1, 16):
                pos = jnp.minimum(pos, pw[i])
            posB = jnp.full((16,), pos, dtype=_I32)
            idm = jnp.full((16,), BIGI, dtype=_I32)
            for ci in range(NCH):
                vi = ids_v[pl.ds(ci * 16, 16)]
                posv = lanes + ci * 16
                idm = jnp.minimum(idm, jnp.where(posv == posB, vi, BIGI))
            pid = idm[0]
            for i in range(1, 16):
                pid = jnp.minimum(pid, idm[i])
            g = pos // G
            slot = pos - g * G
            gv = jnp.full((16,), g, dtype=_I32)
            pidv = jnp.full((16,), pid, dtype=_I32)
            slotv = jnp.full((16,), slot, dtype=_I32)
            lo = jnp.full((16,), g * G, dtype=_I32)
            hi = jnp.full((16,), g * G + G, dtype=_I32)
            for ci in range(NCH):
                v = vals_v[pl.ds(ci * 16, 16)]
                vi = ids_v[pl.ds(ci * 16, 16)]
                posv = lanes + ci * 16
                kill = (vi == pidv) | ((posv >= lo) & (posv < hi))
                vals_v[pl.ds(ci * 16, 16)] = jnp.where(kill, jnp.inf, v)
            a0 = sp_v[pl.ds(0, 16)]
            sp_v[pl.ds(0, 16)] = jnp.where(lanes == gv, slotv, a0)
            a1 = sp_v[pl.ds(16, 16)]
            sp_v[pl.ds(16, 16)] = jnp.where((lanes + 16) == gv, slotv, a1)
            p0 = sp_v[pl.ds(32, 16)]
            sp_v[pl.ds(32, 16)] = jnp.where(lanes == gv, pidv, p0)
            p1 = sp_v[pl.ds(48, 16)]
            sp_v[pl.ds(48, 16)] = jnp.where((lanes + 16) == gv, pidv, p1)
            return carry

        lax.fori_loop(0, G, iter_body, 0)
        pltpu.sync_copy(sp_v, sp_out.at[pl.ds(w * 64, 64)])


def _greedy_call(vals_flat, ids_flat, B, G):
    mesh = plsc.VectorSubcoreMesh(core_axis_name="c", subcore_axis_name="s")
    body = functools.partial(_greedy_sc_body, B, G)
    kern = pl.kernel(
        body,
        out_type=jax.ShapeDtypeStruct((B * 64,), _I32),
        mesh=mesh,
        scratch_types=[pltpu.VMEM((G * G,), _F32),
                       pltpu.VMEM((G * G,), _I32),
                       pltpu.VMEM((64,), _I32)],
    )
    return kern(vals_flat, ids_flat)


def _final_body(B, G, S, sp_ref, preds_ref, gtb_ref, lbl_ref, fneg_ref,
                out_ref, acc):
    N = S * S
    E = 4 + _NUM_CLASSES
    stride = _IMG_SIZE / float(S)
    i = pl.program_id(0)
    blk = preds_ref[0]
    sp = sp_ref[0]
    slot = sp[0:G, 0:1]
    pid = sp[32:32 + G, 0:1]
    valid = slot >= 0
    lbl = lbl_ref[0]

    oh = (lax.broadcasted_iota(_I32, (G, N), 1) == pid).astype(_F32)
    rows = lax.dot_general(oh, blk, (((1,), (0,)), ((), ())),
                           preferred_element_type=_F32)
    tx = rows[:, 0:1]
    ty = rows[:, 1:2]
    tw_r = rows[:, 2:3]
    th_r = rows[:, 3:4]
    ohl = (lax.broadcasted_iota(_I32, (G, E), 1) == (lbl + 4)).astype(_F32)
    msel = jnp.sum(rows * ohl, axis=1, keepdims=True)

    gxf = (pid % S).astype(_F32)
    gyf = (pid // S).astype(_F32)
    px = (_sigmoid(tx) * 2.0 - 0.5 + gxf) * stride
    py = (_sigmoid(ty) * 2.0 - 0.5 + gyf) * stride
    pwd = _softplus(tw_r) * stride
    phd = _softplus(th_r) * stride
    mx1 = px - 0.5 * pwd
    mx2 = px + 0.5 * pwd
    my1 = py - 0.5 * phd
    my2 = py + 0.5 * phd

    gtb = gtb_ref[0]
    gcx = gtb[:, 0:1] * _IMG_SIZE
    gcy = gtb[:, 1:2] * _IMG_SIZE
    gw = gtb[:, 2:3] * _IMG_SIZE
    gh = gtb[:, 3:4] * _IMG_SIZE
    gx1 = gcx - 0.5 * gw
    gx2 = gcx + 0.5 * gw
    gy1 = gcy - 0.5 * gh
    gy2 = gcy + 0.5 * gh

    pw = jnp.maximum(mx2 - mx1, _EPS)
    ph = jnp.maximum(my2 - my1, _EPS)
    tw = jnp.maximum(gx2 - gx1, _EPS)
    th = jnp.maximum(gy2 - gy1, _EPS)
    iw = jnp.maximum(jnp.minimum(mx2, gx2) - jnp.maximum(mx1, gx1), 0.0)
    ih = jnp.maximum(jnp.minimum(my2, gy2) - jnp.maximum(my1, gy1), 0.0)
    inter = iw * ih
    union = pw * ph + tw * th - inter + _EPS
    iou = inter / union
    pcx = (mx1 + mx2) * 0.5
    pcy = (my1 + my2) * 0.5
    tcx = (gx1 + gx2) * 0.5
    tcy = (gy1 + gy2) * 0.5
    cd = (pcx - tcx) ** 2 + (pcy - tcy) ** 2
    cw = jnp.maximum(mx2, gx2) - jnp.minimum(mx1, gx1)
    ch = jnp.maximum(my2, gy2) - jnp.minimum(my1, gy1)
    c2 = cw ** 2 + ch ** 2 + _EPS
    v = (4.0 / (math.pi ** 2)) * (_atan_pos(tw / th) - _atan_pos(pw / ph)) ** 2
    alpha = v / (v - iou + 1.0 + _EPS)
    ciou = iou - cd / c2 - alpha * v
    box_i = jnp.sum(jnp.where(valid, 1.0 - ciou, 0.0))

    lm = msel
    pm = _sigmoid(lm)
    ce0 = _softplus(lm)
    ce1 = ce0 - lm
    delta = _ALPHA * ce1 * (1.0 - pm) ** 2 - (1.0 - _ALPHA) * ce0 * pm ** 2
    cls_i = fneg_ref[0][0, 0] + jnp.sum(jnp.where(valid, delta, 0.0))
    m_i = jnp.sum(jnp.where(valid, 1.0, 0.0))

    @pl.when(i == 0)
    def _():
        acc[0] = 0.0
        acc[1] = 0.0
        acc[2] = 0.0

    acc[0] += box_i
    acc[1] += cls_i
    acc[2] += m_i

    @pl.when(i == B - 1)
    def _():
        denom = jnp.maximum(acc[2], 1.0)
        loss = (_LAMBDA_BOX * acc[0] + _LAMBDA_CLS * acc[1]) / denom
        out_ref[...] = jnp.full((1, 128), loss, dtype=_F32)


def _final_call(sp3, preds_r, gt_boxes, lbl_r, fneg, S):
    B, G, _ = gt_boxes.shape
    N, E = preds_r.shape[1], preds_r.shape[2]
    body = functools.partial(_final_body, B, G, S)
    return pl.pallas_call(
        body,
        grid=(B,),
        in_specs=[pl.BlockSpec((1, 64, 1), lambda i: (i, 0, 0)),
                  pl.BlockSpec((1, N, E), lambda i: (i, 0, 0)),
                  pl.BlockSpec((1, G, 4), lambda i: (i, 0, 0)),
                  pl.BlockSpec((1, G, 1), lambda i: (i, 0, 0)),
                  pl.BlockSpec((1, 1, 128), lambda i: (i, 0, 0))],
        out_specs=pl.BlockSpec((1, 128), lambda i: (0, 0)),
        out_shape=jax.ShapeDtypeStruct((1, 128), _F32),
        scratch_shapes=[pltpu.SMEM((8,), _F32)],
    )(sp3, preds_r, gt_boxes, lbl_r, fneg)


def kernel(preds, gt_boxes, gt_labels):
    B, A, S, S2, E4 = preds.shape
    N = A * S * S2
    E = E4
    G = gt_boxes.shape[1]
    preds_r = preds.reshape(B, N, E)
    lbl_r = gt_labels.astype(_I32).reshape(B, G, 1)
    vals, ids, fneg = _head_call(preds_r, gt_boxes, lbl_r)
    sp = _greedy_call(vals.reshape(-1), ids.reshape(-1), B, G)
    out = _final_call(sp.reshape(B, 64, 1), preds_r, gt_boxes, lbl_r, fneg, S)
    return out[0, 0]
